# baseline (speedup 1.0000x reference)
import jax
import jax.numpy as jnp
from jax import lax
from jax.experimental import pallas as pl
from jax.experimental.pallas import tpu as pltpu
from jax.experimental.pallas import tpu_sc as plsc

N = 100000
D = 128
G = 2048
L = 16
NC = 2
NS = 16
NW = NC * NS
RPT = 3200
NPAD = RPT * NW
SB = 128
NSB = RPT // SB
GRP = SB // L
STRIPE = G // NS
KD = D // L


def _sc_body(x_hbm, b_hbm, wv_hbm, bb_hbm, z_hbm,
             wout_hbm, part_hbm,
             xb0, xb1, ix0, ix1, six0, six1, sb0, sb1, wb0, wb1,
             wvbuf, bbuf, zbuf, acc_sh,
             sx0, sx1, si0, si1, so0, so1, sw0, sw1, sz):
    c = lax.axis_index("c")
    s = lax.axis_index("s")
    wid = c * NS + s
    base = wid * RPT

    bufs = ((xb0, ix0, six0, sb0, wb0, sx0, si0, so0, sw0),
            (xb1, ix1, six1, sb1, wb1, sx1, si1, so1, sw1))

    def start_in(sb, buf):
        xb, ix = buf[0], buf[1]
        sx, si = buf[5], buf[6]
        pltpu.async_copy(x_hbm.at[pl.ds((base + sb * SB) * D, SB * D)],
                         xb, sx)
        pltpu.async_copy(b_hbm.at[pl.ds(base + sb * SB, SB)], ix, si)

    pltpu.async_copy(z_hbm.at[pl.ds(s * STRIPE, STRIPE)],
                     acc_sh.at[pl.ds(s * STRIPE, STRIPE)], sz)
    start_in(0, bufs[0])
    start_in(1, bufs[1])
    pltpu.sync_copy(wv_hbm, wvbuf)
    pltpu.sync_copy(bb_hbm, bbuf)
    pltpu.make_async_copy(z_hbm.at[pl.ds(s * STRIPE, STRIPE)],
                          acc_sh.at[pl.ds(s * STRIPE, STRIPE)], sz).wait()
    plsc.subcore_barrier()

    lanes = lax.iota(jnp.int32, L)
    lane0 = lanes == 0
    bvec = bbuf[...]
    wregs = [wvbuf[pl.ds(k * L, L)] for k in range(KD)]

    def process(sb, buf, first=False):
        xb, ix, six, sbuf, wb = buf[0], buf[1], buf[2], buf[3], buf[4]
        sx, si, so, sw = buf[5], buf[6], buf[7], buf[8]
        idslice = pl.ds(base + sb * SB, SB)
        pltpu.make_async_copy(
            x_hbm.at[pl.ds((base + sb * SB) * D, SB * D)], xb, sx).wait()
        pltpu.make_async_copy(b_hbm.at[idslice], ix, si).wait()
        if not first:
            pltpu.make_async_copy(wb, wout_hbm.at[idslice], sw).wait()
            pltpu.make_async_copy(sbuf, acc_sh.at[six], so).wait()
        for k in range(SB // L):
            six[pl.ds(k * L, L)] = ix[pl.ds(k * L, L)]

        def group(g, _):
            for r in range(L):
                row = g * L + r
                p = xb[pl.ds(row * D, L)] * wregs[0]
                for k in range(1, KD):
                    p = p + xb[pl.ds(row * D + k * L, L)] * wregs[k]
                zr = plsc.cumsum(p)[L - 1]
                plsc.store_scatter(zbuf, [jnp.full((L,), r, jnp.int32)],
                                   jnp.full((L,), zr, jnp.float32),
                                   mask=lane0)
            z = zbuf[...] + bvec
            wgt = 1.0 / (1.0 + jnp.exp(-z))
            wb[pl.ds(g * L, L)] = wgt
            for r in range(L):
                row = g * L + r
                wr = wgt[r]
                for k in range(KD):
                    sbuf[row, pl.ds(k * L, L)] = (
                        xb[pl.ds(row * D + k * L, L)] * wr)
            return 0
        lax.fori_loop(0, GRP, group, 0)

        pltpu.async_copy(wb, wout_hbm.at[idslice], sw)
        pltpu.async_copy(sbuf, acc_sh.at[six], so, add=True)

    process(0, bufs[0], first=True)
    start_in(2, bufs[0])
    process(1, bufs[1], first=True)
    start_in(3, bufs[1])

    def pair(p, _):
        process(2 * p, bufs[0])
        start_in(2 * p + 2, bufs[0])
        process(2 * p + 1, bufs[1])
        start_in(2 * p + 3, bufs[1])
        return 0
    lax.fori_loop(1, 11, pair, 0)

    process(22, bufs[0])
    start_in(24, bufs[0])
    process(23, bufs[1])
    process(24, bufs[0])

    pltpu.make_async_copy(sb0, acc_sh.at[six0], so0).wait()
    pltpu.make_async_copy(sb1, acc_sh.at[six1], so1).wait()
    pltpu.make_async_copy(wb0, wout_hbm.at[pl.ds(base + 24 * SB, SB)],
                          sw0).wait()
    pltpu.make_async_copy(wb1, wout_hbm.at[pl.ds(base + 23 * SB, SB)],
                          sw1).wait()
    plsc.subcore_barrier()
    pltpu.sync_copy(acc_sh.at[pl.ds(s * STRIPE, STRIPE)],
                    part_hbm.at[c, pl.ds(s * STRIPE, STRIPE)])


_sc_call = pl.kernel(
    _sc_body,
    mesh=plsc.VectorSubcoreMesh(core_axis_name="c", subcore_axis_name="s"),
    compiler_params=pltpu.CompilerParams(needs_layout_passes=False),
    out_type=[jax.ShapeDtypeStruct((NPAD,), jnp.float32),
              jax.ShapeDtypeStruct((NC, G, D), jnp.float32)],
    scratch_types=[
        pltpu.VMEM((SB * D,), jnp.float32),
        pltpu.VMEM((SB * D,), jnp.float32),
        pltpu.VMEM((SB,), jnp.int32),
        pltpu.VMEM((SB,), jnp.int32),
        pltpu.VMEM((SB,), jnp.int32),
        pltpu.VMEM((SB,), jnp.int32),
        pltpu.VMEM((SB, D), jnp.float32),
        pltpu.VMEM((SB, D), jnp.float32),
        pltpu.VMEM((SB,), jnp.float32),
        pltpu.VMEM((SB,), jnp.float32),
        pltpu.VMEM((D,), jnp.float32),
        pltpu.VMEM((L,), jnp.float32),
        pltpu.VMEM((L,), jnp.float32),
        pltpu.VMEM_SHARED((G, D), jnp.float32),
        pltpu.SemaphoreType.DMA,
        pltpu.SemaphoreType.DMA,
        pltpu.SemaphoreType.DMA,
        pltpu.SemaphoreType.DMA,
        pltpu.SemaphoreType.DMA,
        pltpu.SemaphoreType.DMA,
        pltpu.SemaphoreType.DMA,
        pltpu.SemaphoreType.DMA,
        pltpu.SemaphoreType.DMA,
    ],
)


def _merge_body(p_ref, o_ref):
    o_ref[...] = p_ref[0] + p_ref[1]


def _merge(p):
    return pl.pallas_call(
        _merge_body,
        out_shape=jax.ShapeDtypeStruct((G, D), jnp.float32),
    )(p)


def kernel(x, batch, W, b):
    xp = jnp.pad(x, ((0, NPAD - N), (0, 0))).reshape(NPAD * D)
    bp = jnp.pad(batch.astype(jnp.int32), (0, NPAD - N),
                 constant_values=G - 1)
    zeros = jnp.zeros((G, D), jnp.float32)
    wout, part = _sc_call(
        xp, bp, W[:, 0], jnp.full((L,), b[0], jnp.float32), zeros)
    hg = _merge(part)
    return hg, wout[:N].reshape(N, 1)


# speedup vs baseline: 1.2372x; 1.2372x over previous
import jax
import jax.numpy as jnp
from jax import lax
from jax.experimental import pallas as pl
from jax.experimental.pallas import tpu as pltpu
from jax.experimental.pallas import tpu_sc as plsc

N = 100000
D = 128
G = 2048
L = 16
NC = 2
NS = 16
NW = NC * NS
RPT = 3200
NPAD = RPT * NW
SB = 128
NSB = RPT // SB
GRP = SB // L
STRIPE = G // NS
KD = D // L
NX = N % SB
NP = SB - NX


def _sc_body(x_hbm, xp_hbm, b_hbm, wv_hbm, bb_hbm, z_hbm,
             wout_hbm, part_hbm,
             xb0, xb1, ix0, ix1, six0, six1, sb0, sb1, wb0, wb1,
             wvbuf, bbuf, zbuf, acc_sh,
             sx0, sx1, si0, si1, so0, so1, sw0, sw1, sz):
    c = lax.axis_index("c")
    s = lax.axis_index("s")
    wid = c * NS + s
    base = wid * RPT

    bufs = ((xb0, ix0, six0, sb0, wb0, sx0, si0, so0, sw0),
            (xb1, ix1, six1, sb1, wb1, sx1, si1, so1, sw1))

    def start_in(sb, buf):
        xb, ix = buf[0], buf[1]
        sx, si = buf[5], buf[6]
        gr0 = base + sb * SB
        @pl.when(gr0 + SB <= N)
        def _():
            pltpu.async_copy(x_hbm.at[pl.ds(gr0 * D, SB * D)], xb, sx)

        @pl.when(gr0 >= N)
        def _():
            pltpu.async_copy(xp_hbm.at[pl.ds((gr0 - N) * D, SB * D)],
                             xb, sx)

        @pl.when((gr0 < N) & (gr0 + SB > N))
        def _():
            pltpu.async_copy(x_hbm.at[pl.ds(gr0 * D, NX * D)],
                             xb.at[pl.ds(0, NX * D)], sx)
            pltpu.async_copy(xp_hbm.at[pl.ds(0, NP * D)],
                             xb.at[pl.ds(NX * D, NP * D)], sx)
        pltpu.async_copy(b_hbm.at[pl.ds(gr0, SB)], ix, si)

    pltpu.async_copy(z_hbm.at[pl.ds(s * STRIPE, STRIPE)],
                     acc_sh.at[pl.ds(s * STRIPE, STRIPE)], sz)
    start_in(0, bufs[0])
    start_in(1, bufs[1])
    pltpu.sync_copy(wv_hbm, wvbuf)
    pltpu.sync_copy(bb_hbm, bbuf)
    pltpu.make_async_copy(z_hbm.at[pl.ds(s * STRIPE, STRIPE)],
                          acc_sh.at[pl.ds(s * STRIPE, STRIPE)], sz).wait()
    plsc.subcore_barrier()

    lanes = lax.iota(jnp.int32, L)
    lane0 = lanes == 0
    bvec = bbuf[...]
    wregs = [wvbuf[pl.ds(k * L, L)] for k in range(KD)]

    def process(sb, buf, first=False):
        xb, ix, six, sbuf, wb = buf[0], buf[1], buf[2], buf[3], buf[4]
        sx, si, so, sw = buf[5], buf[6], buf[7], buf[8]
        idslice = pl.ds(base + sb * SB, SB)
        pltpu.make_async_copy(x_hbm.at[pl.ds(0, SB * D)], xb, sx).wait()
        pltpu.make_async_copy(b_hbm.at[idslice], ix, si).wait()
        if not first:
            pltpu.make_async_copy(wb, wout_hbm.at[idslice], sw).wait()
            pltpu.make_async_copy(sbuf, acc_sh.at[six], so).wait()
        for k in range(SB // L):
            six[pl.ds(k * L, L)] = ix[pl.ds(k * L, L)]

        def group(g, _):
            for r in range(L):
                row = g * L + r
                p = xb[pl.ds(row * D, L)] * wregs[0]
                for k in range(1, KD):
                    p = p + xb[pl.ds(row * D + k * L, L)] * wregs[k]
                zr = plsc.cumsum(p)[L - 1]
                plsc.store_scatter(zbuf, [jnp.full((L,), r, jnp.int32)],
                                   jnp.full((L,), zr, jnp.float32),
                                   mask=lane0)
            z = zbuf[...] + bvec
            wgt = 1.0 / (1.0 + jnp.exp(-z))
            wb[pl.ds(g * L, L)] = wgt
            for r in range(L):
                row = g * L + r
                wr = wgt[r]
                for k in range(KD):
                    sbuf[row, pl.ds(k * L, L)] = (
                        xb[pl.ds(row * D + k * L, L)] * wr)
            return 0
        lax.fori_loop(0, GRP, group, 0)

        pltpu.async_copy(wb, wout_hbm.at[idslice], sw)
        pltpu.async_copy(sbuf, acc_sh.at[six], so, add=True)

    process(0, bufs[0], first=True)
    start_in(2, bufs[0])
    process(1, bufs[1], first=True)
    start_in(3, bufs[1])

    def pair(p, _):
        process(2 * p, bufs[0])
        start_in(2 * p + 2, bufs[0])
        process(2 * p + 1, bufs[1])
        start_in(2 * p + 3, bufs[1])
        return 0
    lax.fori_loop(1, 11, pair, 0)

    process(22, bufs[0])
    start_in(24, bufs[0])
    process(23, bufs[1])
    process(24, bufs[0])

    pltpu.make_async_copy(sb0, acc_sh.at[six0], so0).wait()
    pltpu.make_async_copy(sb1, acc_sh.at[six1], so1).wait()
    pltpu.make_async_copy(wb0, wout_hbm.at[pl.ds(base + 24 * SB, SB)],
                          sw0).wait()
    pltpu.make_async_copy(wb1, wout_hbm.at[pl.ds(base + 23 * SB, SB)],
                          sw1).wait()
    plsc.subcore_barrier()
    pltpu.sync_copy(acc_sh.at[pl.ds(s * STRIPE, STRIPE)],
                    part_hbm.at[c, pl.ds(s * STRIPE, STRIPE)])


_sc_call = pl.kernel(
    _sc_body,
    mesh=plsc.VectorSubcoreMesh(core_axis_name="c", subcore_axis_name="s"),
    compiler_params=pltpu.CompilerParams(needs_layout_passes=False),
    out_type=[jax.ShapeDtypeStruct((NPAD,), jnp.float32),
              jax.ShapeDtypeStruct((NC, G, D), jnp.float32)],
    scratch_types=[
        pltpu.VMEM((SB * D,), jnp.float32),
        pltpu.VMEM((SB * D,), jnp.float32),
        pltpu.VMEM((SB,), jnp.int32),
        pltpu.VMEM((SB,), jnp.int32),
        pltpu.VMEM((SB,), jnp.int32),
        pltpu.VMEM((SB,), jnp.int32),
        pltpu.VMEM((SB, D), jnp.float32),
        pltpu.VMEM((SB, D), jnp.float32),
        pltpu.VMEM((SB,), jnp.float32),
        pltpu.VMEM((SB,), jnp.float32),
        pltpu.VMEM((D,), jnp.float32),
        pltpu.VMEM((L,), jnp.float32),
        pltpu.VMEM((L,), jnp.float32),
        pltpu.VMEM_SHARED((G + 1, D), jnp.float32),
        pltpu.SemaphoreType.DMA,
        pltpu.SemaphoreType.DMA,
        pltpu.SemaphoreType.DMA,
        pltpu.SemaphoreType.DMA,
        pltpu.SemaphoreType.DMA,
        pltpu.SemaphoreType.DMA,
        pltpu.SemaphoreType.DMA,
        pltpu.SemaphoreType.DMA,
        pltpu.SemaphoreType.DMA,
    ],
)


def _merge_body(p_ref, o_ref):
    o_ref[...] = p_ref[0] + p_ref[1]


def _merge(p):
    return pl.pallas_call(
        _merge_body,
        out_shape=jax.ShapeDtypeStruct((G, D), jnp.float32),
    )(p)


def kernel(x, batch, W, b):
    xf = x.reshape(N * D)
    xpad = jnp.zeros(((NPAD - N) * D,), jnp.float32)
    bp = jnp.pad(batch.astype(jnp.int32), (0, NPAD - N),
                 constant_values=G)
    zeros = jnp.zeros((G, D), jnp.float32)
    wout, part = _sc_call(
        xf, xpad, bp, W[:, 0], jnp.full((L,), b[0], jnp.float32), zeros)
    hg = _merge(part)
    return hg, wout[:N].reshape(N, 1)


# speedup vs baseline: 2.0244x; 1.6363x over previous
import jax
import jax.numpy as jnp
from jax import lax
from jax.experimental import pallas as pl
from jax.experimental.pallas import tpu as pltpu
from jax.experimental.pallas import tpu_sc as plsc

N = 100000
D = 128
G = 2048
L = 16
NC = 2
NS = 16
NW = NC * NS
RPT = 3200
NPAD = RPT * NW
SB = 128
NSB = RPT // SB
GRP = SB // L
STRIPE = G // NS
KD = D // L
NX = N % SB
NP = SB - NX


def _sc_body(x_hbm, xp_hbm, b_hbm, wv_hbm, bb_hbm, z_hbm,
             wout_hbm, part_hbm,
             xb0, xb1, ix0, ix1, six0, six1, sb0, sb1, wb0, wb1,
             wvbuf, bbuf, zbuf, acc_sh,
             sx0, sx1, si0, si1, so0, so1, sw0, sw1, sz):
    c = lax.axis_index("c")
    s = lax.axis_index("s")
    wid = c * NS + s
    base = wid * RPT

    bufs = ((xb0, ix0, six0, sb0, wb0, sx0, si0, so0, sw0),
            (xb1, ix1, six1, sb1, wb1, sx1, si1, so1, sw1))

    def start_in(sb, buf):
        xb, ix = buf[0], buf[1]
        sx, si = buf[5], buf[6]
        gr0 = base + sb * SB
        @pl.when(gr0 + SB <= N)
        def _():
            pltpu.async_copy(x_hbm.at[pl.ds(gr0 * D, SB * D)], xb, sx)

        @pl.when(gr0 >= N)
        def _():
            pltpu.async_copy(xp_hbm.at[pl.ds((gr0 - N) * D, SB * D)],
                             xb, sx)

        @pl.when((gr0 < N) & (gr0 + SB > N))
        def _():
            pltpu.async_copy(x_hbm.at[pl.ds(gr0 * D, NX * D)],
                             xb.at[pl.ds(0, NX * D)], sx)
            pltpu.async_copy(xp_hbm.at[pl.ds(0, NP * D)],
                             xb.at[pl.ds(NX * D, NP * D)], sx)
        pltpu.async_copy(b_hbm.at[pl.ds(gr0, SB)], ix, si)

    pltpu.async_copy(z_hbm.at[pl.ds(s * STRIPE, STRIPE)],
                     acc_sh.at[pl.ds(s * STRIPE, STRIPE)], sz)
    start_in(0, bufs[0])
    start_in(1, bufs[1])
    pltpu.sync_copy(wv_hbm, wvbuf)
    pltpu.sync_copy(bb_hbm, bbuf)
    pltpu.make_async_copy(z_hbm.at[pl.ds(s * STRIPE, STRIPE)],
                          acc_sh.at[pl.ds(s * STRIPE, STRIPE)], sz).wait()
    plsc.subcore_barrier()

    lanes = lax.iota(jnp.int32, L)
    lane0 = lanes == 0
    bvec = bbuf[...]
    wregs = [wvbuf[pl.ds(k * L, L)] for k in range(KD)]

    def process(sb, buf, first=False):
        xb, ix, six, sbuf, wb = buf[0], buf[1], buf[2], buf[3], buf[4]
        sx, si, so, sw = buf[5], buf[6], buf[7], buf[8]
        idslice = pl.ds(base + sb * SB, SB)
        pltpu.make_async_copy(x_hbm.at[pl.ds(0, SB * D)], xb, sx).wait()
        pltpu.make_async_copy(b_hbm.at[idslice], ix, si).wait()
        if not first:
            pltpu.make_async_copy(wb, wout_hbm.at[idslice], sw).wait()
            pltpu.make_async_copy(sbuf, acc_sh.at[six], so).wait()
        for k in range(SB // L):
            six[pl.ds(k * L, L)] = ix[pl.ds(k * L, L)]

        def group(g, _):
            z = bvec
            for r in range(L):
                row = g * L + r
                p = xb[pl.ds(row * D, L)] * wregs[0]
                for k in range(1, KD):
                    p = p + xb[pl.ds(row * D + k * L, L)] * wregs[k]
                zr = plsc.cumsum(p)[L - 1]
                z = jnp.where(lanes == r, zr, z)
            wgt = 1.0 / (1.0 + jnp.exp(-z))
            wb[pl.ds(g * L, L)] = wgt
            for r in range(L):
                row = g * L + r
                wr = wgt[r]
                for k in range(KD):
                    sbuf[row, pl.ds(k * L, L)] = (
                        xb[pl.ds(row * D + k * L, L)] * wr)
            return 0
        lax.fori_loop(0, GRP, group, 0)

        pltpu.async_copy(wb, wout_hbm.at[idslice], sw)
        pltpu.async_copy(sbuf, acc_sh.at[six], so, add=True)

    process(0, bufs[0], first=True)
    start_in(2, bufs[0])
    process(1, bufs[1], first=True)
    start_in(3, bufs[1])

    def pair(p, _):
        process(2 * p, bufs[0])
        start_in(2 * p + 2, bufs[0])
        process(2 * p + 1, bufs[1])
        start_in(2 * p + 3, bufs[1])
        return 0
    lax.fori_loop(1, 11, pair, 0)

    process(22, bufs[0])
    start_in(24, bufs[0])
    process(23, bufs[1])
    process(24, bufs[0])

    pltpu.make_async_copy(sb0, acc_sh.at[six0], so0).wait()
    pltpu.make_async_copy(sb1, acc_sh.at[six1], so1).wait()
    pltpu.make_async_copy(wb0, wout_hbm.at[pl.ds(base + 24 * SB, SB)],
                          sw0).wait()
    pltpu.make_async_copy(wb1, wout_hbm.at[pl.ds(base + 23 * SB, SB)],
                          sw1).wait()
    plsc.subcore_barrier()
    pltpu.sync_copy(acc_sh.at[pl.ds(s * STRIPE, STRIPE)],
                    part_hbm.at[c, pl.ds(s * STRIPE, STRIPE)])


_sc_call = pl.kernel(
    _sc_body,
    mesh=plsc.VectorSubcoreMesh(core_axis_name="c", subcore_axis_name="s"),
    compiler_params=pltpu.CompilerParams(needs_layout_passes=False),
    out_type=[jax.ShapeDtypeStruct((NPAD,), jnp.float32),
              jax.ShapeDtypeStruct((NC, G, D), jnp.float32)],
    scratch_types=[
        pltpu.VMEM((SB * D,), jnp.float32),
        pltpu.VMEM((SB * D,), jnp.float32),
        pltpu.VMEM((SB,), jnp.int32),
        pltpu.VMEM((SB,), jnp.int32),
        pltpu.VMEM((SB,), jnp.int32),
        pltpu.VMEM((SB,), jnp.int32),
        pltpu.VMEM((SB, D), jnp.float32),
        pltpu.VMEM((SB, D), jnp.float32),
        pltpu.VMEM((SB,), jnp.float32),
        pltpu.VMEM((SB,), jnp.float32),
        pltpu.VMEM((D,), jnp.float32),
        pltpu.VMEM((L,), jnp.float32),
        pltpu.VMEM((L,), jnp.float32),
        pltpu.VMEM_SHARED((G + 1, D), jnp.float32),
        pltpu.SemaphoreType.DMA,
        pltpu.SemaphoreType.DMA,
        pltpu.SemaphoreType.DMA,
        pltpu.SemaphoreType.DMA,
        pltpu.SemaphoreType.DMA,
        pltpu.SemaphoreType.DMA,
        pltpu.SemaphoreType.DMA,
        pltpu.SemaphoreType.DMA,
        pltpu.SemaphoreType.DMA,
    ],
)


def _merge_body(p_ref, o_ref):
    o_ref[...] = p_ref[0] + p_ref[1]


def _merge(p):
    return pl.pallas_call(
        _merge_body,
        out_shape=jax.ShapeDtypeStruct((G, D), jnp.float32),
    )(p)


def kernel(x, batch, W, b):
    xf = x.reshape(N * D)
    xpad = jnp.zeros(((NPAD - N) * D,), jnp.float32)
    bp = jnp.pad(batch.astype(jnp.int32), (0, NPAD - N),
                 constant_values=G)
    zeros = jnp.zeros((G, D), jnp.float32)
    wout, part = _sc_call(
        xf, xpad, bp, W[:, 0], jnp.full((L,), b[0], jnp.float32), zeros)
    hg = _merge(part)
    return hg, wout[:N].reshape(N, 1)
